# baseline (speedup 1.0000x reference)
import functools

import jax
import jax.numpy as jnp
from jax import lax
from jax.experimental import pallas as pl
from jax.experimental.pallas import tpu as pltpu
from jax.experimental.pallas import tpu_sc as plsc

_IOU_THR = 0.5
_ROIS = 256
_SCORE_THR = -1e9
_LANES = 128

_C_LO = 350
_C_HI = 700
_BIS_ITERS = 28
_CAP_W = 384
_REGION = 512


def _prep_body(s_ref, dst_ref, cnt_ref, n, parts, batch):
    rows = s_ref.shape[1]
    rows_pc = rows // parts
    s = s_ref[0]
    lanei = lax.broadcasted_iota(jnp.int32, (rows, _LANES), 1)
    rowi = lax.broadcasted_iota(jnp.int32, (rows, _LANES), 0)
    iota = rowi * _LANES + lanei
    real = iota < n
    hi0 = jnp.max(jnp.where(real, s, -jnp.inf))
    lo0 = jnp.min(jnp.where(real, s, jnp.inf))
    c_hi0 = jnp.sum((s >= hi0).astype(jnp.int32))

    def it(_, carry):
        lo, hi, tb, cb = carry
        mid = 0.5 * (lo + hi)
        c = jnp.sum((s >= mid).astype(jnp.int32))
        better = jnp.logical_and(c <= _C_HI, c > cb)
        tb = jnp.where(better, mid, tb)
        cb = jnp.where(better, c, cb)
        go_up = c > _C_HI
        lo = jnp.where(go_up, mid, lo)
        hi = jnp.where(go_up, hi, mid)
        return lo, hi, tb, cb

    _, _, tb, _ = lax.fori_loop(0, _BIS_ITERS, it, (lo0, hi0, hi0, c_hi0))

    cand = jnp.logical_and(s >= tb, s > _SCORE_THR)
    m = jnp.where(cand, 1, 0)
    c_in = m
    for sh in (1, 2, 4, 8, 16, 32, 64):
        rolled = jnp.roll(c_in, sh, axis=1)
        c_in = c_in + jnp.where(lanei >= sh, rolled, 0)
    rs0 = lax.slice(c_in, (0, _LANES - 1), (rows, _LANES))
    rowi1 = lax.broadcasted_iota(jnp.int32, (rows, 1), 0)
    rc = rs0
    sh = 1
    while sh < rows:
        rolled = jnp.roll(rc, sh, axis=0)
        rc = rc + jnp.where(rowi1 >= sh, rolled, 0)
        sh *= 2
    pfx = c_in + (rc - rs0)

    start = jnp.zeros((rows, 1), jnp.int32)
    prev = jnp.int32(0)
    for p in range(parts):
        sp = prev
        endv = rc[rows_pc * (p + 1) - 1, 0]
        start = jnp.where(rowi1 >= rows_pc * p, sp, start)
        cnt_ref[0, 0, p] = endv - sp
        prev = endv

    dst_rel = pfx - 1 - start
    trash = _CAP_W + lanei
    dst = jnp.where(jnp.logical_and(cand, dst_rel < _CAP_W), dst_rel, trash)
    bid = pl.program_id(0)
    regbase = jnp.zeros((rows, 1), jnp.int32)
    for p in range(parts):
        regbase = jnp.where(rowi1 >= rows_pc * p,
                            (bid * parts + p) * _REGION, regbase)
    dst_ref[0] = dst + regbase


def _sc_move_body(nc, parts, npad, rows_pc, nb,
                  s_h, x_h, y_h, w_h, h_h, dst_h,
                  cs_h, cx_h, cy_h, cw_h, ch_h, ci_h,
                  dstv, idxv, sv, xv, yv, wv, hv, sem):
    wid = lax.axis_index("s") * nc + lax.axis_index("c")

    @pl.when(wid < nb * parts)
    def _():
        _sc_move_one(nc, parts, npad, rows_pc, wid,
                     s_h, x_h, y_h, w_h, h_h, dst_h,
                     cs_h, cx_h, cy_h, cw_h, ch_h, ci_h,
                     dstv, idxv, sv, xv, yv, wv, hv, sem)


def _sc_move_one(nc, parts, npad, rows_pc, wid,
                 s_h, x_h, y_h, w_h, h_h, dst_h,
                 cs_h, cx_h, cy_h, cw_h, ch_h, ci_h,
                 dstv, idxv, sv, xv, yv, wv, hv, sem):
    b = wid // parts
    p = wid - b * parts
    chunk = rows_pc * _LANES
    gbase = b * npad + p * chunk
    gsl = pl.ds(gbase, chunk)
    pltpu.sync_copy(dst_h.at[gsl], dstv)
    pltpu.sync_copy(s_h.at[gsl], sv)
    pltpu.sync_copy(x_h.at[gsl], xv)
    pltpu.sync_copy(y_h.at[gsl], yv)
    pltpu.sync_copy(w_h.at[gsl], wv)
    pltpu.sync_copy(h_h.at[gsl], hv)

    iota16 = lax.iota(jnp.int32, 16)
    for off16 in range(0, chunk, 16):
        idxv[pl.ds(off16, 16)] = iota16 + (gbase + off16)
    copies = [pltpu.async_copy(idxv, ci_h.at[dstv], sem),
              pltpu.async_copy(sv, cs_h.at[dstv], sem),
              pltpu.async_copy(xv, cx_h.at[dstv], sem),
              pltpu.async_copy(yv, cy_h.at[dstv], sem),
              pltpu.async_copy(wv, cw_h.at[dstv], sem),
              pltpu.async_copy(hv, ch_h.at[dstv], sem)]
    for cp in copies:
        cp.wait()


def _sc_compact(s2d, x2d, y2d, w2d, h2d, dst2d, b, npad, parts, nc):
    nw = b * parts
    rows_pc = (npad // _LANES) // parts
    f32 = jnp.float32
    i32 = jnp.int32
    sck = functools.partial(
        pl.kernel,
        mesh=plsc.VectorSubcoreMesh(core_axis_name="c", subcore_axis_name="s"),
        out_type=[jax.ShapeDtypeStruct((nw * _REGION,), f32)] * 5
        + [jax.ShapeDtypeStruct((nw * _REGION,), i32)],
        scratch_types=[pltpu.VMEM((rows_pc * _LANES,), i32),
                       pltpu.VMEM((rows_pc * _LANES,), i32)]
        + [pltpu.VMEM((rows_pc * _LANES,), f32)] * 5
        + [pltpu.SemaphoreType.DMA],
    )(functools.partial(_sc_move_body, nc, parts, npad, rows_pc, b))
    return sck(s2d, x2d, y2d, w2d, h2d, dst2d)


def _dense_batch(b, s_ref, x_ref, y_ref, w_ref, h_ref, out_ref,
                 x1f, y1f, x2f, y2f, a3f):
    rows = s_ref.shape[1]
    x = x_ref[b]
    y = y_ref[b]
    w = w_ref[b]
    h = h_ref[b]
    ws = jnp.floor(w * 0.5)
    hs = jnp.floor(h * 0.5)
    x1 = x - ws
    x2 = x + ws
    y1 = y - hs
    y2 = y + hs
    x1f[...] = x1
    y1f[...] = y1
    x2f[...] = x2
    y2f[...] = y2
    a3f[...] = (x2 - x1) * (y2 - y1) * (1.0 / 3.0)
    s = s_ref[b]
    msc0 = jnp.where(s > _SCORE_THR, s, -jnp.inf)

    iota = (lax.broadcasted_iota(jnp.int32, (rows, _LANES), 0) * _LANES
            + lax.broadcasted_iota(jnp.int32, (rows, _LANES), 1))
    lane = lax.broadcasted_iota(jnp.int32, (1, _LANES), 1)
    big = jnp.int32(rows * _LANES)

    def step(k, msc):
        m = jnp.max(msc)
        idx = jnp.min(jnp.where(msc == m, iota, big))
        has = m > -jnp.inf
        row = idx // _LANES
        col = idx - row * _LANES
        pick = lane == col
        bx1 = jnp.sum(jnp.where(pick, x1f[pl.ds(row, 1), :], 0.0))
        by1 = jnp.sum(jnp.where(pick, y1f[pl.ds(row, 1), :], 0.0))
        bx2 = jnp.sum(jnp.where(pick, x2f[pl.ds(row, 1), :], 0.0))
        by2 = jnp.sum(jnp.where(pick, y2f[pl.ds(row, 1), :], 0.0))
        ba3 = jnp.sum(jnp.where(pick, a3f[pl.ds(row, 1), :], 0.0))
        ix1 = jnp.maximum(x1f[...], bx1)
        iy1 = jnp.maximum(y1f[...], by1)
        ix2 = jnp.minimum(x2f[...], bx2)
        iy2 = jnp.minimum(y2f[...], by2)
        inter = jnp.maximum(ix2 - ix1, 0.0) * jnp.maximum(iy2 - iy1, 0.0)
        kill = jnp.logical_and(inter > (a3f[...] + ba3), has)
        out_ref[b, 0, k] = jnp.where(has, idx, jnp.int32(-1))
        return jnp.where(kill, -jnp.inf, msc)

    lax.fori_loop(0, _ROIS, step, msc0)


def _greedy_body(cs_ref, cx_ref, cy_ref, cw_ref, ch_ref, ci_ref, cnt_ref,
                 s_ref, x_ref, y_ref, w_ref, h_ref, out_ref,
                 x1f, y1f, x2f, y2f, a3f, nb, parts, npad):
    rows_per_part = _REGION // _LANES
    crow = rows_per_part * parts
    pos = (lax.broadcasted_iota(jnp.int32, (crow, _LANES), 0) * _LANES
           + lax.broadcasted_iota(jnp.int32, (crow, _LANES), 1))
    rowi = lax.broadcasted_iota(jnp.int32, (crow, _LANES), 0)
    posm = ((rowi % rows_per_part) * _LANES
            + lax.broadcasted_iota(jnp.int32, (crow, _LANES), 1))
    big = jnp.int32(crow * _LANES)

    x1s, y1s, x2s, y2s, a3s, cis, mscs, valids = [], [], [], [], [], [], [], []
    for b in range(nb):
        cx = cx_ref[b]
        cy = cy_ref[b]
        cw = cw_ref[b]
        ch = ch_ref[b]
        ws = jnp.floor(cw * 0.5)
        hs = jnp.floor(ch * 0.5)
        x1 = cx - ws
        x2 = cx + ws
        y1 = cy - hs
        y2 = cy + hs
        x1s.append(x1)
        y1s.append(y1)
        x2s.append(x2)
        y2s.append(y2)
        a3s.append((x2 - x1) * (y2 - y1) * (1.0 / 3.0))
        cis.append(ci_ref[b])
        cnts = [cnt_ref[b, 0, p] for p in range(parts)]
        valid = cnts[0] <= _CAP_W
        for p in range(1, parts):
            valid = jnp.logical_and(valid, cnts[p] <= _CAP_W)
        valids.append(valid)
        climit = cnts[parts - 1]
        for p in range(parts - 2, -1, -1):
            climit = jnp.where(rowi < (p + 1) * rows_per_part, cnts[p], climit)
        cs = cs_ref[b]
        vmask = jnp.logical_and(posm < climit, cs > _SCORE_THR)
        mscs.append(jnp.where(vmask, cs, -jnp.inf))

    def step(k, carry):
        mscs, comps = carry
        mscs_n, comps_n = [], []
        for b in range(nb):
            msc = mscs[b]
            m = jnp.max(msc)
            idx = jnp.min(jnp.where(msc == m, pos, big))
            has = m > -jnp.inf
            pick = pos == idx
            oid = jnp.sum(jnp.where(pick, cis[b], 0))
            bx1 = jnp.sum(jnp.where(pick, x1s[b], 0.0))
            by1 = jnp.sum(jnp.where(pick, y1s[b], 0.0))
            bx2 = jnp.sum(jnp.where(pick, x2s[b], 0.0))
            by2 = jnp.sum(jnp.where(pick, y2s[b], 0.0))
            ba3 = jnp.sum(jnp.where(pick, a3s[b], 0.0))
            ix1 = jnp.maximum(x1s[b], bx1)
            iy1 = jnp.maximum(y1s[b], by1)
            ix2 = jnp.minimum(x2s[b], bx2)
            iy2 = jnp.minimum(y2s[b], by2)
            inter = jnp.maximum(ix2 - ix1, 0.0) * jnp.maximum(iy2 - iy1, 0.0)
            kill = jnp.logical_and(inter > (a3s[b] + ba3), has)
            out_ref[b, 0, k] = jnp.where(has, oid - b * npad, jnp.int32(-1))
            mscs_n.append(jnp.where(kill, -jnp.inf, msc))
            comps_n.append(jnp.logical_and(comps[b], has))
        return mscs_n, comps_n

    comps0 = [jnp.bool_(True)] * nb
    _, comps = lax.fori_loop(0, _ROIS, step, (mscs, comps0))

    for b in range(nb):
        ok = jnp.logical_and(valids[b], comps[b])

        @pl.when(jnp.logical_not(ok))
        def _():
            _dense_batch(b, s_ref, x_ref, y_ref, w_ref, h_ref, out_ref,
                         x1f, y1f, x2f, y2f, a3f)


def kernel(input):
    b, n, _ = input.shape
    info = plsc.get_sparse_core_info()
    nc, ns = info.num_cores, info.num_subcores
    parts = max(1, (nc * ns) // b)
    align = parts * 8
    rows = (n + _LANES - 1) // _LANES
    rows = ((rows + align - 1) // align) * align
    npad = rows * _LANES
    pad = npad - n

    s = jnp.pad(input[:, :, 0], ((0, 0), (0, pad)), constant_values=-jnp.inf)
    x = jnp.pad(input[:, :, 1], ((0, 0), (0, pad)))
    y = jnp.pad(input[:, :, 2], ((0, 0), (0, pad)))
    w = jnp.pad(input[:, :, 3], ((0, 0), (0, pad)))
    h = jnp.pad(input[:, :, 4], ((0, 0), (0, pad)))
    shape3 = (b, rows, _LANES)
    s3, x3, y3, w3, h3 = (a.reshape(shape3) for a in (s, x, y, w, h))

    spec = pl.BlockSpec((1, rows, _LANES), lambda i: (i, 0, 0))
    dst, cnt = pl.pallas_call(
        functools.partial(_prep_body, n=n, parts=parts, batch=b),
        grid=(b,),
        in_specs=[spec],
        out_specs=[pl.BlockSpec((1, rows, _LANES), lambda i: (i, 0, 0)),
                   pl.BlockSpec((1, 1, 16), lambda i: (i, 0, 0),
                                memory_space=pltpu.SMEM)],
        out_shape=[jax.ShapeDtypeStruct((b, rows, _LANES), jnp.int32),
                   jax.ShapeDtypeStruct((b, 1, 16), jnp.int32)],
        compiler_params=pltpu.CompilerParams(
            dimension_semantics=("arbitrary",)),
    )(s3)

    cs, cx, cy, cw, ch, ci = _sc_compact(
        s3.reshape(-1), x3.reshape(-1), y3.reshape(-1), w3.reshape(-1),
        h3.reshape(-1), dst.reshape(-1), b, npad, parts, nc)
    capb = parts * _REGION

    crow = capb // _LANES
    cs, cx, cy, cw, ch = (a.reshape(b, crow, _LANES)
                          for a in (cs, cx, cy, cw, ch))
    ci = ci.reshape(b, crow, _LANES)
    cspec = pl.BlockSpec((b, crow, _LANES), lambda: (0, 0, 0))
    fspec = pl.BlockSpec((b, rows, _LANES), lambda: (0, 0, 0))
    sels = pl.pallas_call(
        functools.partial(_greedy_body, nb=b, parts=parts, npad=npad),
        grid=(),
        in_specs=[cspec] * 6
        + [pl.BlockSpec((b, 1, 16), lambda: (0, 0, 0),
                        memory_space=pltpu.SMEM)]
        + [fspec] * 5,
        out_specs=pl.BlockSpec((b, 1, _ROIS), lambda: (0, 0, 0),
                               memory_space=pltpu.SMEM),
        out_shape=jax.ShapeDtypeStruct((b, 1, _ROIS), jnp.int32),
        scratch_shapes=[pltpu.VMEM((rows, _LANES), jnp.float32)] * 5,
    )(cs, cx, cy, cw, ch, ci, cnt, s3, x3, y3, w3, h3)
    sels = sels.reshape(b, _ROIS)

    keys = jax.random.split(jax.random.key(1), b)
    rand = jax.vmap(
        lambda k: jax.random.randint(k, (_ROIS,), 0, n, dtype=jnp.int32))(keys)
    return jnp.where(sels >= 0, sels, rand)


# speedup vs baseline: 6.5194x; 6.5194x over previous
import jax
import jax.numpy as jnp
from jax.experimental import pallas as pl
from jax.experimental.pallas import tpu as pltpu

_IOU_THR = 0.5
_ROIS = 256
_SCORE_THR = -1e9
_LANES = 128


def _nms_body(s_ref, x_ref, y_ref, w_ref, h_ref, out_ref,
              x1_ref, y1_ref, x2_ref, y2_ref, a3_ref):
    rows = s_ref.shape[1]
    x = x_ref[0]
    y = y_ref[0]
    w = w_ref[0]
    h = h_ref[0]
    ws = jnp.floor(w * 0.5)
    hs = jnp.floor(h * 0.5)
    x1 = x - ws
    x2 = x + ws
    y1 = y - hs
    y2 = y + hs
    x1_ref[...] = x1
    y1_ref[...] = y1
    x2_ref[...] = x2
    y2_ref[...] = y2
    a3_ref[...] = (x2 - x1) * (y2 - y1) * (1.0 / 3.0)
    s = s_ref[0]
    msc0 = jnp.where(s > _SCORE_THR, s, -jnp.inf)

    iota = (jax.lax.broadcasted_iota(jnp.int32, (rows, _LANES), 0) * _LANES
            + jax.lax.broadcasted_iota(jnp.int32, (rows, _LANES), 1))
    lane = jax.lax.broadcasted_iota(jnp.int32, (1, _LANES), 1)
    big = jnp.int32(rows * _LANES)

    def step(k, msc):
        m = jnp.max(msc)
        idx = jnp.min(jnp.where(msc == m, iota, big))
        has = m > -jnp.inf
        row = idx // _LANES
        col = idx - row * _LANES
        pick = lane == col
        bx1 = jnp.sum(jnp.where(pick, x1_ref[pl.ds(row, 1), :], 0.0))
        by1 = jnp.sum(jnp.where(pick, y1_ref[pl.ds(row, 1), :], 0.0))
        bx2 = jnp.sum(jnp.where(pick, x2_ref[pl.ds(row, 1), :], 0.0))
        by2 = jnp.sum(jnp.where(pick, y2_ref[pl.ds(row, 1), :], 0.0))
        ba3 = jnp.sum(jnp.where(pick, a3_ref[pl.ds(row, 1), :], 0.0))
        ix1 = jnp.maximum(x1_ref[...], bx1)
        iy1 = jnp.maximum(y1_ref[...], by1)
        ix2 = jnp.minimum(x2_ref[...], bx2)
        iy2 = jnp.minimum(y2_ref[...], by2)
        inter = jnp.maximum(ix2 - ix1, 0.0) * jnp.maximum(iy2 - iy1, 0.0)
        kill = jnp.logical_and(inter > (a3_ref[...] + ba3), has)
        out_ref[0, 0, k] = jnp.where(has, idx, jnp.int32(-1))
        return jnp.where(kill, -jnp.inf, msc)

    jax.lax.fori_loop(0, _ROIS, step, msc0)


def kernel(input):
    b, n, _ = input.shape
    rows = (n + _LANES - 1) // _LANES
    npad = rows * _LANES
    pad = npad - n

    s = jnp.pad(input[:, :, 0], ((0, 0), (0, pad)), constant_values=-jnp.inf)
    x = jnp.pad(input[:, :, 1], ((0, 0), (0, pad)))
    y = jnp.pad(input[:, :, 2], ((0, 0), (0, pad)))
    w = jnp.pad(input[:, :, 3], ((0, 0), (0, pad)))
    h = jnp.pad(input[:, :, 4], ((0, 0), (0, pad)))
    shape3 = (b, rows, _LANES)
    s, x, y, w, h = (a.reshape(shape3) for a in (s, x, y, w, h))

    spec = pl.BlockSpec((1, rows, _LANES), lambda i: (i, 0, 0))
    sels = pl.pallas_call(
        _nms_body,
        grid=(b,),
        in_specs=[spec] * 5,
        out_specs=pl.BlockSpec(
            (1, 1, _ROIS), lambda i: (i, 0, 0), memory_space=pltpu.SMEM),
        out_shape=jax.ShapeDtypeStruct((b, 1, _ROIS), jnp.int32),
        scratch_shapes=[pltpu.VMEM((rows, _LANES), jnp.float32)] * 5,
        compiler_params=pltpu.CompilerParams(
            dimension_semantics=("parallel",)),
    )(s, x, y, w, h)
    sels = sels.reshape(b, _ROIS)

    keys = jax.random.split(jax.random.key(1), b)
    rand = jax.vmap(
        lambda k: jax.random.randint(k, (_ROIS,), 0, n, dtype=jnp.int32))(keys)
    return jnp.where(sels >= 0, sels, rand)
